# baseline (speedup 1.0000x reference)
import functools

import jax
import jax.numpy as jnp
from jax import lax
from jax.experimental import pallas as pl
from jax.experimental.pallas import tpu as pltpu
from jax.experimental.pallas import tpu_sc as plsc

_N = 10000
_E = 320000
_D = 128

_NC = 2
_NS = 16
_NW = _NC * _NS
_EPW = _E // _NW
_K = 128
_EPP = 10240
_CH = _EPP // _K
_NP = 10240
_RPT = _NP // _NS
_NBUF = 2
_NPASS = 2
_HCH = _CH // _NPASS


_ROWS = 2000


def _lin3_body(x_ref, w_ref, Wxv_ref, bxv_ref, Wwv_ref, bwv_ref,
               Wwe_ref, bwe_ref, x1_ref, w1_ref, w2_ref):
    x = x_ref[...]
    w = w_ref[...]
    x1_ref[...] = jnp.dot(x, Wxv_ref[...],
                          preferred_element_type=jnp.float32) + bxv_ref[...]
    w1_ref[...] = jnp.dot(w, Wwv_ref[...],
                          preferred_element_type=jnp.float32) + bwv_ref[...]
    w2_ref[...] = jnp.dot(w, Wwe_ref[...],
                          preferred_element_type=jnp.float32) + bwe_ref[...]


def _update_lin_body(x1_ref, p0_ref, p1_ref, Wxe_ref, bxe_ref,
                     xn_ref, x2_ref):
    x1 = x1_ref[...]
    xn = x1 + (p0_ref[...] + p1_ref[...]) * x1
    xn_ref[...] = xn
    x2_ref[...] = jnp.dot(xn, Wxe_ref[...],
                          preferred_element_type=jnp.float32) + bxe_ref[...]


def _update_body(w2_ref, q0_ref, q1_ref, wn_ref):
    w2 = w2_ref[...]
    wn_ref[...] = w2 + (q0_ref[...] + q1_ref[...]) * w2


def _row_spec():
    return pl.BlockSpec((_ROWS, _D), lambda i: (i, 0))


def _full_spec(shape):
    return pl.BlockSpec(shape, lambda i: tuple(0 for _ in shape))


def _sc_aggregate(table, gidx, sidx):
    mesh = plsc.VectorSubcoreMesh(core_axis_name="c", subcore_axis_name="s")

    @functools.partial(
        pl.kernel,
        mesh=mesh,
        out_type=jax.ShapeDtypeStruct((_NC * _NP, _D), jnp.float32),
        scratch_types=[
            pltpu.VMEM((_HCH, _K), jnp.int32),
            pltpu.VMEM((_HCH, _K), jnp.int32),
            pltpu.VMEM((_NBUF * _K, _D), jnp.float32),
            pltpu.SemaphoreType.DMA,
            pltpu.SemaphoreType.DMA,
            pltpu.VMEM_SHARED((_NP, _D), jnp.float32),
        ],
    )
    def body(table_hbm, gidx_hbm, sidx_hbm, out_hbm,
             gidx_v, sidx_v, big_v, gsem, ssem, accum_sh):
        c = lax.axis_index("c")
        s = lax.axis_index("s")
        wid = s * _NC + c

        def zrow(i, _):
            for j in range(_D // 16):
                big_v[i, pl.ds(j * 16, 16)] = jnp.zeros((16,), jnp.float32)
            return _
        lax.fori_loop(0, _K, zrow, None)
        for r in range(_RPT // _K):
            pltpu.sync_copy(
                big_v.at[pl.ds(0, _K)],
                accum_sh.at[pl.ds(s * _RPT + r * _K, _K)])
        plsc.subcore_barrier()

        def bufslot(j):
            off = pl.multiple_of(lax.rem(j, _NBUF) * _K, _K)
            return big_v.at[pl.ds(off, _K)]

        def fire_gather(j):
            pltpu.async_copy(table_hbm.at[gidx_v.at[j]], bufslot(j), gsem)

        for p in range(_NPASS):
            pltpu.sync_copy(gidx_hbm.at[wid * _NPASS + p], gidx_v)
            pltpu.sync_copy(sidx_hbm.at[wid * _NPASS + p], sidx_v)

            def step(j, _):
                rows = big_v.at[pl.ds(0, _K)]
                pltpu.async_copy(
                    table_hbm.at[gidx_v.at[j]], rows, gsem).wait()
                pltpu.sync_copy(rows, accum_sh.at[sidx_v.at[j]], add=True)
                return _
            lax.fori_loop(0, _HCH, step, None)
        plsc.subcore_barrier()

        pltpu.sync_copy(accum_sh.at[pl.ds(s * _RPT, _RPT)],
                        out_hbm.at[pl.ds(c * _NP + s * _RPT, _RPT)])

    return body(table, gidx, sidx)


def kernel(h, x, w, Wx_v, bx_v, Ww_v, bw_v, Wx_e, bx_e, Ww_e, bw_e):
    h32 = h.astype(jnp.int32)
    pad_g = jnp.zeros((_NW, _EPP - _EPW), jnp.int32)
    pad_s = jnp.full((_NW, _EPP - _EPW), _NP - 1, jnp.int32)
    src_w = h32[0].reshape(_NW, _EPW)
    dst_w = h32[1].reshape(_NW, _EPW)

    def _padded(a, pad):
        return jnp.concatenate([a, pad], axis=1).reshape(
            _NW * _NPASS, _HCH, _K)

    src_g, src_s = _padded(src_w, pad_g), _padded(src_w, pad_s)
    dst_g, dst_s = _padded(dst_w, pad_g), _padded(dst_w, pad_s)
    bx_v2 = bx_v.reshape(1, _D)
    bw_v2 = bw_v.reshape(1, _D)
    bx_e2 = bx_e.reshape(1, _D)
    bw_e2 = bw_e.reshape(1, _D)

    wspec = _full_spec((_D, _D))
    bspec = _full_spec((1, _D))

    x1, w1, w2 = pl.pallas_call(
        _lin3_body,
        grid=(_N // _ROWS,),
        in_specs=[_row_spec(), _row_spec(),
                  wspec, bspec, wspec, bspec, wspec, bspec],
        out_specs=[_row_spec(), _row_spec(), _row_spec()],
        out_shape=[jax.ShapeDtypeStruct((_N, _D), jnp.float32)] * 3,
    )(x, w, Wx_v, bx_v2, Ww_v, bw_v2, Ww_e, bw_e2)

    pv = _sc_aggregate(w1, dst_g, src_s)

    x_new, x2 = pl.pallas_call(
        _update_lin_body,
        grid=(_N // _ROWS,),
        in_specs=[_row_spec(), _row_spec(), _row_spec(), wspec, bspec],
        out_specs=[_row_spec(), _row_spec()],
        out_shape=[jax.ShapeDtypeStruct((_N, _D), jnp.float32)] * 2,
    )(x1, pv[:_N], pv[_NP:_NP + _N], Wx_e, bx_e2)

    qv = _sc_aggregate(x2, src_g, dst_s)

    w_new = pl.pallas_call(
        _update_body,
        grid=(_N // _ROWS,),
        in_specs=[_row_spec(), _row_spec(), _row_spec()],
        out_specs=_row_spec(),
        out_shape=jax.ShapeDtypeStruct((_N, _D), jnp.float32),
    )(w2, qv[:_N], qv[_NP:_NP + _N])

    return (w_new, x_new)


# speedup vs baseline: 3.2229x; 3.2229x over previous
import functools

import jax
import jax.numpy as jnp
from jax import lax
from jax.experimental import pallas as pl
from jax.experimental.pallas import tpu as pltpu
from jax.experimental.pallas import tpu_sc as plsc

_N = 10000
_E = 320000
_D = 128

_NC = 2
_NS = 16
_NW = _NC * _NS
_EPW = _E // _NW
_K = 80
_EPP = _EPW
_CH = _EPP // _K
_NP = 10240
_RPT = _NP // _NS
_NBUF = 2
_NPASS = 1
_HCH = _CH // _NPASS


_ROWS = 2000


def _lin3_body(x_ref, w_ref, Wxv_ref, bxv_ref, Wwv_ref, bwv_ref,
               Wwe_ref, bwe_ref, x1_ref, w1_ref, w2_ref):
    x = x_ref[...]
    w = w_ref[...]
    x1_ref[...] = jnp.dot(x, Wxv_ref[...],
                          preferred_element_type=jnp.float32) + bxv_ref[...]
    w1_ref[...] = jnp.dot(w, Wwv_ref[...],
                          preferred_element_type=jnp.float32) + bwv_ref[...]
    w2_ref[...] = jnp.dot(w, Wwe_ref[...],
                          preferred_element_type=jnp.float32) + bwe_ref[...]


def _update_lin_body(x1_ref, p0_ref, p1_ref, Wxe_ref, bxe_ref,
                     xn_ref, x2_ref):
    x1 = x1_ref[...]
    xn = x1 + (p0_ref[...] + p1_ref[...]) * x1
    xn_ref[...] = xn
    x2_ref[...] = jnp.dot(xn, Wxe_ref[...],
                          preferred_element_type=jnp.float32) + bxe_ref[...]


def _update_body(w2_ref, q0_ref, q1_ref, wn_ref):
    w2 = w2_ref[...]
    wn_ref[...] = w2 + (q0_ref[...] + q1_ref[...]) * w2


def _row_spec():
    return pl.BlockSpec((_ROWS, _D), lambda i: (i, 0))


def _full_spec(shape):
    return pl.BlockSpec(shape, lambda i: tuple(0 for _ in shape))


def _sc_aggregate(table, gidx, sidx):
    mesh = plsc.VectorSubcoreMesh(core_axis_name="c", subcore_axis_name="s")

    @functools.partial(
        pl.kernel,
        mesh=mesh,
        out_type=jax.ShapeDtypeStruct((_NC * _NP, _D), jnp.float32),
        scratch_types=[
            pltpu.VMEM((_EPP,), jnp.int32),
            pltpu.VMEM((_HCH, _K), jnp.int32),
            pltpu.VMEM((_NBUF * _K, _D), jnp.float32),
            pltpu.SemaphoreType.DMA,
            pltpu.SemaphoreType.DMA,
            pltpu.VMEM_SHARED((_NP, _D), jnp.float32),
        ],
    )
    def body(table_hbm, gidx_hbm, sidx_hbm, out_hbm,
             gidx_v, sidx_v, big_v, gsem, ssem, accum_sh):
        c = lax.axis_index("c")
        s = lax.axis_index("s")
        wid = s * _NC + c

        def zrow(i, _):
            for j in range(_D // 16):
                big_v[i, pl.ds(j * 16, 16)] = jnp.zeros((16,), jnp.float32)
            return _
        lax.fori_loop(0, _K, zrow, None)
        for r in range(_RPT // _K):
            pltpu.sync_copy(
                big_v.at[pl.ds(0, _K)],
                accum_sh.at[pl.ds(s * _RPT + r * _K, _K)])
        plsc.subcore_barrier()

        def bufslot(j):
            off = pl.multiple_of(lax.rem(j, _NBUF) * _K, _K)
            return big_v.at[pl.ds(off, _K)]

        def gslice(j):
            off = pl.multiple_of(j * _K, 8)
            return gidx_v.at[pl.ds(off, _K)]

        def fire_gather(j):
            pltpu.async_copy(table_hbm.at[gslice(j)], bufslot(j), gsem)

        for p in range(_NPASS):
            pltpu.sync_copy(
                gidx_hbm.at[pl.ds(wid * _EPP, _EPP)], gidx_v)
            pltpu.sync_copy(sidx_hbm.at[wid * _NPASS + p], sidx_v)
            fire_gather(0)

            def step(j, _):
                @pl.when(j + 1 < _HCH)
                def _fire():
                    fire_gather(j + 1)
                pltpu.make_async_copy(
                    table_hbm.at[gidx_v.at[pl.ds(0, _K)]],
                    big_v.at[pl.ds(0, _K)], gsem).wait()
                pltpu.async_copy(
                    bufslot(j), accum_sh.at[sidx_v.at[j]], ssem, add=True)
                pltpu.make_async_copy(
                    big_v.at[pl.ds(0, _K)], accum_sh.at[sidx_v.at[j]],
                    ssem).wait()
                return _
            lax.fori_loop(0, _HCH, step, None)
        plsc.subcore_barrier()

        pltpu.sync_copy(accum_sh.at[pl.ds(s * _RPT, _RPT)],
                        out_hbm.at[pl.ds(c * _NP + s * _RPT, _RPT)])

    return body(table, gidx, sidx)


def kernel(h, x, w, Wx_v, bx_v, Ww_v, bw_v, Wx_e, bx_e, Ww_e, bw_e):
    h32 = h.astype(jnp.int32)
    pad_g = jnp.zeros((_NW, _EPP - _EPW), jnp.int32)
    pad_s = jnp.full((_NW, _EPP - _EPW), _NP - 1, jnp.int32)
    src_w = h32[0].reshape(_NW, _EPW)
    dst_w = h32[1].reshape(_NW, _EPW)

    def _padded(a, pad):
        return jnp.concatenate([a, pad], axis=1).reshape(
            _NW * _NPASS, _HCH, _K)

    src_g = jnp.concatenate([src_w, pad_g], axis=1).reshape(-1)
    dst_g = jnp.concatenate([dst_w, pad_g], axis=1).reshape(-1)
    src_s = _padded(src_w, pad_s)
    dst_s = _padded(dst_w, pad_s)
    bx_v2 = bx_v.reshape(1, _D)
    bw_v2 = bw_v.reshape(1, _D)
    bx_e2 = bx_e.reshape(1, _D)
    bw_e2 = bw_e.reshape(1, _D)

    wspec = _full_spec((_D, _D))
    bspec = _full_spec((1, _D))

    x1, w1, w2 = pl.pallas_call(
        _lin3_body,
        grid=(_N // _ROWS,),
        in_specs=[_row_spec(), _row_spec(),
                  wspec, bspec, wspec, bspec, wspec, bspec],
        out_specs=[_row_spec(), _row_spec(), _row_spec()],
        out_shape=[jax.ShapeDtypeStruct((_N, _D), jnp.float32)] * 3,
    )(x, w, Wx_v, bx_v2, Ww_v, bw_v2, Ww_e, bw_e2)

    pv = _sc_aggregate(w1, dst_g, src_s)

    x_new, x2 = pl.pallas_call(
        _update_lin_body,
        grid=(_N // _ROWS,),
        in_specs=[_row_spec(), _row_spec(), _row_spec(), wspec, bspec],
        out_specs=[_row_spec(), _row_spec()],
        out_shape=[jax.ShapeDtypeStruct((_N, _D), jnp.float32)] * 2,
    )(x1, pv[:_N], pv[_NP:_NP + _N], Wx_e, bx_e2)

    qv = _sc_aggregate(x2, src_g, dst_s)

    w_new = pl.pallas_call(
        _update_body,
        grid=(_N // _ROWS,),
        in_specs=[_row_spec(), _row_spec(), _row_spec()],
        out_specs=_row_spec(),
        out_shape=jax.ShapeDtypeStruct((_N, _D), jnp.float32),
    )(w2, qv[:_N], qv[_NP:_NP + _N])

    return (w_new, x_new)
